# baseline (speedup 1.0000x reference)
import functools

import jax
import jax.numpy as jnp
from jax import lax
from jax.experimental import pallas as pl
from jax.experimental.pallas import tpu as pltpu
from jax.experimental.pallas import tpu_sc as plsc

N = 10000
E = 320000
D_IN = 128
D_EDGE = 16
D_HALF = 64
D_BASE = 192
D_OUT = 128

NC = 2
NS = 16
NW = NC * NS
EDGES_TILE = E // NW

CHA = 200
CHUNKS_A = EDGES_TILE // CHA
CHC = 80
CHUNKS_C = EDGES_TILE // CHC
CHUNKS_C_MAIN = CHUNKS_C - 1

_mesh = plsc.VectorSubcoreMesh(
    core_axis_name="c", subcore_axis_name="s", num_cores=NC, num_subcores=NS)


def _phase_a_body(src_h, dst_h, ea_h, ones_h, z16_h, z8_h,
                  pn_out, deg_out,
                  pn_sh, deg_sh,
                  sidx0, sidx1, didx0, didx1, attr0, attr1, ones_v,
                  asem0, asem1, isem0, isem1):
    c = lax.axis_index("c")
    s = lax.axis_index("s")

    @pl.when(s < 10)
    def _():
        r0 = s * 1000
        pltpu.sync_copy(z16_h.at[pl.ds(r0, 1000)], pn_sh.at[pl.ds(r0, 1000)])
        pltpu.sync_copy(z8_h.at[pl.ds(r0, 1000)], deg_sh.at[pl.ds(r0, 1000)])

    wid = c * NS + s
    ebase = wid * EDGES_TILE
    pltpu.sync_copy(ones_h, ones_v)
    plsc.subcore_barrier()

    pltpu.async_copy(src_h.at[wid, 0], sidx0, isem0)
    pltpu.async_copy(dst_h.at[wid, 0], didx0, isem0)
    pltpu.async_copy(src_h.at[wid, 1], sidx1, isem1)
    pltpu.async_copy(dst_h.at[wid, 1], didx1, isem1)
    pltpu.async_copy(ea_h.at[pl.ds(ebase, CHA)], attr0, asem0)
    pltpu.async_copy(ea_h.at[pl.ds(ebase + CHA, CHA)], attr1, asem1)

    @pl.loop(0, CHUNKS_A, step=2)
    def _(i):
        off = ebase + i * CHA
        pltpu.make_async_copy(ea_h.at[pl.ds(off, CHA)], attr0, asem0).wait()
        pltpu.make_async_copy(src_h.at[wid, i], sidx0, isem0).wait()
        pltpu.make_async_copy(dst_h.at[wid, i], didx0, isem0).wait()
        pltpu.sync_copy(attr0, pn_sh.at[sidx0], add=True)
        pltpu.sync_copy(ones_v, deg_sh.at[didx0], add=True)

        @pl.when(i + 2 < CHUNKS_A)
        def _():
            pltpu.async_copy(src_h.at[wid, i + 2], sidx0, isem0)
            pltpu.async_copy(dst_h.at[wid, i + 2], didx0, isem0)
            pltpu.async_copy(ea_h.at[pl.ds(off + 2 * CHA, CHA)], attr0, asem0)

        pltpu.make_async_copy(ea_h.at[pl.ds(off + CHA, CHA)], attr1, asem1).wait()
        pltpu.make_async_copy(src_h.at[wid, i + 1], sidx1, isem1).wait()
        pltpu.make_async_copy(dst_h.at[wid, i + 1], didx1, isem1).wait()
        pltpu.sync_copy(attr1, pn_sh.at[sidx1], add=True)
        pltpu.sync_copy(ones_v, deg_sh.at[didx1], add=True)

        @pl.when(i + 3 < CHUNKS_A)
        def _():
            pltpu.async_copy(src_h.at[wid, i + 3], sidx1, isem1)
            pltpu.async_copy(dst_h.at[wid, i + 3], didx1, isem1)
            pltpu.async_copy(ea_h.at[pl.ds(off + 3 * CHA, CHA)], attr1, asem1)

    plsc.subcore_barrier()

    @pl.when(s < 10)
    def _():
        r0 = s * 1000
        pltpu.sync_copy(pn_sh.at[pl.ds(r0, 1000)], pn_out.at[c, pl.ds(r0, 1000)])
        pltpu.sync_copy(deg_sh.at[pl.ds(r0, 1000)], deg_out.at[c, pl.ds(r0, 1000)])


_phase_a = functools.partial(
    pl.kernel,
    out_type=(jax.ShapeDtypeStruct((NC, N, D_EDGE), jnp.float32),
              jax.ShapeDtypeStruct((NC, N, 8), jnp.float32)),
    mesh=_mesh,
    scratch_types=(
        pltpu.VMEM_SHARED((N, D_EDGE), jnp.float32),
        pltpu.VMEM_SHARED((N, 8), jnp.float32),
        pltpu.VMEM((CHA,), jnp.int32),
        pltpu.VMEM((CHA,), jnp.int32),
        pltpu.VMEM((CHA,), jnp.int32),
        pltpu.VMEM((CHA,), jnp.int32),
        pltpu.VMEM((CHA, D_EDGE), jnp.float32),
        pltpu.VMEM((CHA, D_EDGE), jnp.float32),
        pltpu.VMEM((CHA, 8), jnp.float32),
        pltpu.SemaphoreType.DMA,
        pltpu.SemaphoreType.DMA,
        pltpu.SemaphoreType.DMA,
        pltpu.SemaphoreType.DMA,
    ),
)(_phase_a_body)


def _phase_c_body(src_h, dst_h, g_h, z128_h,
                  agg_out,
                  agg_sh,
                  sidx0, sidx1, didx0, didx1, rows0, rows1,
                  gsem0, gsem1, isem0, isem1):
    c = lax.axis_index("c")
    s = lax.axis_index("s")

    @pl.when(s < 10)
    def _():
        r0 = s * 1000
        pltpu.sync_copy(z128_h.at[pl.ds(r0, 1000)], agg_sh.at[pl.ds(r0, 1000)])

    wid = c * NS + s
    plsc.subcore_barrier()

    pltpu.async_copy(src_h.at[wid, 0], sidx0, isem0)
    pltpu.async_copy(dst_h.at[wid, 0], didx0, isem0)
    pltpu.async_copy(src_h.at[wid, 1], sidx1, isem1)
    pltpu.async_copy(dst_h.at[wid, 1], didx1, isem1)
    pltpu.make_async_copy(src_h.at[wid, 0], sidx0, isem0).wait()
    pltpu.make_async_copy(dst_h.at[wid, 0], didx0, isem0).wait()
    pltpu.async_copy(g_h.at[sidx0], rows0, gsem0)

    @pl.loop(0, CHUNKS_C_MAIN, step=2)
    def _(i):
        pltpu.make_async_copy(g_h.at[sidx0], rows0, gsem0).wait()
        pltpu.make_async_copy(src_h.at[wid, i + 1], sidx1, isem1).wait()
        pltpu.make_async_copy(dst_h.at[wid, i + 1], didx1, isem1).wait()
        pltpu.async_copy(g_h.at[sidx1], rows1, gsem1)
        pltpu.sync_copy(rows0, agg_sh.at[didx0], add=True)

        @pl.when(i + 2 < CHUNKS_C)
        def _():
            pltpu.async_copy(src_h.at[wid, i + 2], sidx0, isem0)
            pltpu.async_copy(dst_h.at[wid, i + 2], didx0, isem0)

        pltpu.make_async_copy(g_h.at[sidx1], rows1, gsem1).wait()

        @pl.when(i + 2 < CHUNKS_C)
        def _():
            pltpu.make_async_copy(src_h.at[wid, i + 2], sidx0, isem0).wait()
            pltpu.make_async_copy(dst_h.at[wid, i + 2], didx0, isem0).wait()
            pltpu.async_copy(g_h.at[sidx0], rows0, gsem0)

        pltpu.sync_copy(rows1, agg_sh.at[didx1], add=True)

        @pl.when(i + 3 < CHUNKS_C)
        def _():
            pltpu.async_copy(src_h.at[wid, i + 3], sidx1, isem1)
            pltpu.async_copy(dst_h.at[wid, i + 3], didx1, isem1)

    pltpu.make_async_copy(g_h.at[sidx0], rows0, gsem0).wait()
    pltpu.sync_copy(rows0, agg_sh.at[didx0], add=True)

    plsc.subcore_barrier()

    @pl.when(s < 10)
    def _():
        r0 = s * 1000
        pltpu.sync_copy(agg_sh.at[pl.ds(r0, 1000)], agg_out.at[c, pl.ds(r0, 1000)])


_phase_c = functools.partial(
    pl.kernel,
    out_type=jax.ShapeDtypeStruct((NC, N, D_OUT), jnp.float32),
    mesh=_mesh,
    scratch_types=(
        pltpu.VMEM_SHARED((N, D_OUT), jnp.float32),
        pltpu.VMEM((CHC,), jnp.int32),
        pltpu.VMEM((CHC,), jnp.int32),
        pltpu.VMEM((CHC,), jnp.int32),
        pltpu.VMEM((CHC,), jnp.int32),
        pltpu.VMEM((CHC, D_OUT), jnp.float32),
        pltpu.VMEM((CHC, D_OUT), jnp.float32),
        pltpu.SemaphoreType.DMA,
        pltpu.SemaphoreType.DMA,
        pltpu.SemaphoreType.DMA,
        pltpu.SemaphoreType.DMA,
    ),
)(_phase_c_body)


_RB = 1000


def _dense1_body(x_ref, pn0_ref, pn1_ref, dg0_ref, dg1_ref,
                 We_ref, be_ref, W_ref, Wr_ref, bb_ref,
                 g_ref, base_ref):
    x = x_ref[...]
    pn = pn0_ref[...] + pn1_ref[...]
    deg = dg0_ref[...] + dg1_ref[...]
    dinv = lax.rsqrt(deg[:, :1] + 1.0)
    proj = jnp.dot(pn, We_ref[...], preferred_element_type=jnp.float32)
    proj = proj + be_ref[...]
    proj = jnp.where(proj >= 0, proj, 0.01 * proj)
    xcat = jnp.concatenate([x, proj], axis=1)
    h = jnp.dot(xcat, W_ref[...], preferred_element_type=jnp.float32)
    res = jnp.dot(xcat, Wr_ref[...], preferred_element_type=jnp.float32)
    g_ref[...] = h * dinv
    base_ref[...] = res + bb_ref[...]


def _dense1(x, pn0, pn1, dg0, dg1, We, be2, W, Wr, bb2):
    grid = (N // _RB,)
    return pl.pallas_call(
        _dense1_body,
        grid=grid,
        in_specs=[
            pl.BlockSpec((_RB, D_IN), lambda i: (i, 0)),
            pl.BlockSpec((_RB, D_EDGE), lambda i: (i, 0)),
            pl.BlockSpec((_RB, D_EDGE), lambda i: (i, 0)),
            pl.BlockSpec((_RB, 8), lambda i: (i, 0)),
            pl.BlockSpec((_RB, 8), lambda i: (i, 0)),
            pl.BlockSpec((D_EDGE, D_HALF), lambda i: (0, 0)),
            pl.BlockSpec((1, D_HALF), lambda i: (0, 0)),
            pl.BlockSpec((D_BASE, D_OUT), lambda i: (0, 0)),
            pl.BlockSpec((D_BASE, D_OUT), lambda i: (0, 0)),
            pl.BlockSpec((1, D_OUT), lambda i: (0, 0)),
        ],
        out_specs=[
            pl.BlockSpec((_RB, D_OUT), lambda i: (i, 0)),
            pl.BlockSpec((_RB, D_OUT), lambda i: (i, 0)),
        ],
        out_shape=[
            jax.ShapeDtypeStruct((N, D_OUT), jnp.float32),
            jax.ShapeDtypeStruct((N, D_OUT), jnp.float32),
        ],
    )(x, pn0, pn1, dg0, dg1, We, be2, W, Wr, bb2)


def _dense2_body(agg0_ref, agg1_ref, g_ref, base_ref,
                 dg0_ref, dg1_ref, out_ref):
    deg = dg0_ref[...] + dg1_ref[...]
    dinv = lax.rsqrt(deg[:, :1] + 1.0)
    agg = agg0_ref[...] + agg1_ref[...] + g_ref[...]
    out_ref[...] = jnp.maximum(dinv * agg + base_ref[...], 0.0)


def _dense2(agg0, agg1, g, base, dg0, dg1):
    grid = (N // _RB,)
    bspec = pl.BlockSpec((_RB, D_OUT), lambda i: (i, 0))
    return pl.pallas_call(
        _dense2_body,
        grid=grid,
        in_specs=[
            bspec, bspec, bspec, bspec,
            pl.BlockSpec((_RB, 8), lambda i: (i, 0)),
            pl.BlockSpec((_RB, 8), lambda i: (i, 0)),
        ],
        out_specs=bspec,
        out_shape=jax.ShapeDtypeStruct((N, D_OUT), jnp.float32),
    )(agg0, agg1, g, base, dg0, dg1)


@jax.jit
def _run(x, edge_index, edge_attr, We, be, W, b, Wr, br):
    src = edge_index[0]
    dst = edge_index[1]

    src_a = src.reshape(NW, CHUNKS_A, CHA)
    dst_a = dst.reshape(NW, CHUNKS_A, CHA)
    src_c = src.reshape(NW, CHUNKS_C, CHC)
    dst_c = dst.reshape(NW, CHUNKS_C, CHC)

    ones_v = jnp.ones((CHA, 8), jnp.float32)
    z16 = jnp.zeros((N, D_EDGE), jnp.float32)
    z8 = jnp.zeros((N, 8), jnp.float32)
    z128 = jnp.zeros((N, D_OUT), jnp.float32)

    pn_part, deg_part = _phase_a(src_a, dst_a, edge_attr, ones_v, z16, z8)

    be2 = be.reshape(1, D_HALF)
    bb2 = (b + br).reshape(1, D_OUT)
    g, base = _dense1(x, pn_part[0], pn_part[1], deg_part[0], deg_part[1],
                      We, be2, W, Wr, bb2)

    agg = _phase_c(src_c, dst_c, g, z128)

    out = _dense2(agg[0], agg[1], g, base, deg_part[0], deg_part[1])
    return out


def kernel(x, edge_index, edge_attr, batch, We, be, W, b, Wr, br):
    return _run(x, edge_index, edge_attr, We, be, W, b, Wr, br)


# speedup vs baseline: 1.2059x; 1.2059x over previous
import functools

import jax
import jax.numpy as jnp
from jax import lax
from jax.experimental import pallas as pl
from jax.experimental.pallas import tpu as pltpu
from jax.experimental.pallas import tpu_sc as plsc

N = 10000
E = 320000
D_IN = 128
D_EDGE = 16
D_HALF = 64
D_BASE = 192
D_OUT = 128

NC = 2
NS = 16
NW = NC * NS
EDGES_TILE = E // NW

CHA = 200
CHUNKS_A = EDGES_TILE // CHA
CHC = 80
CHUNKS_C = EDGES_TILE // CHC
CHUNKS_C_MAIN = CHUNKS_C - 1

_mesh = plsc.VectorSubcoreMesh(
    core_axis_name="c", subcore_axis_name="s", num_cores=NC, num_subcores=NS)


def _phase_a_body(src_h, dst_h, ea_h, ones_h, z16_h, z8_h,
                  pn_out, deg_out,
                  pn_sh, deg_sh,
                  sidx0, sidx1, didx0, didx1, attr0, attr1, ones_v,
                  asem0, asem1, isem0, isem1):
    c = lax.axis_index("c")
    s = lax.axis_index("s")

    @pl.when(s < 10)
    def _():
        r0 = s * 1000
        pltpu.sync_copy(z16_h.at[pl.ds(r0, 1000)], pn_sh.at[pl.ds(r0, 1000)])
        pltpu.sync_copy(z8_h.at[pl.ds(r0, 1000)], deg_sh.at[pl.ds(r0, 1000)])

    wid = c * NS + s
    pltpu.sync_copy(ones_h, ones_v)
    plsc.subcore_barrier()

    pltpu.async_copy(src_h.at[wid, 0], sidx0, isem0)
    pltpu.async_copy(dst_h.at[wid, 0], didx0, isem0)
    pltpu.async_copy(src_h.at[wid, 1], sidx1, isem1)
    pltpu.async_copy(dst_h.at[wid, 1], didx1, isem1)
    pltpu.async_copy(ea_h.at[wid, pl.ds(0, CHA)], attr0, asem0)
    pltpu.async_copy(ea_h.at[wid, pl.ds(CHA, CHA)], attr1, asem1)

    @pl.loop(0, CHUNKS_A, step=2)
    def _(i):
        off = pl.multiple_of(i * CHA, 8)
        pltpu.make_async_copy(ea_h.at[wid, pl.ds(off, CHA)], attr0, asem0).wait()
        pltpu.make_async_copy(src_h.at[wid, i], sidx0, isem0).wait()
        pltpu.make_async_copy(dst_h.at[wid, i], didx0, isem0).wait()
        pltpu.sync_copy(attr0, pn_sh.at[sidx0], add=True)
        pltpu.sync_copy(ones_v, deg_sh.at[didx0], add=True)

        @pl.when(i + 2 < CHUNKS_A)
        def _():
            pltpu.async_copy(src_h.at[wid, i + 2], sidx0, isem0)
            pltpu.async_copy(dst_h.at[wid, i + 2], didx0, isem0)
            pltpu.async_copy(ea_h.at[wid, pl.ds(off + 2 * CHA, CHA)], attr0, asem0)

        pltpu.make_async_copy(ea_h.at[wid, pl.ds(off + CHA, CHA)], attr1, asem1).wait()
        pltpu.make_async_copy(src_h.at[wid, i + 1], sidx1, isem1).wait()
        pltpu.make_async_copy(dst_h.at[wid, i + 1], didx1, isem1).wait()
        pltpu.sync_copy(attr1, pn_sh.at[sidx1], add=True)
        pltpu.sync_copy(ones_v, deg_sh.at[didx1], add=True)

        @pl.when(i + 3 < CHUNKS_A)
        def _():
            pltpu.async_copy(src_h.at[wid, i + 3], sidx1, isem1)
            pltpu.async_copy(dst_h.at[wid, i + 3], didx1, isem1)
            pltpu.async_copy(ea_h.at[wid, pl.ds(off + 3 * CHA, CHA)], attr1, asem1)

    plsc.subcore_barrier()

    @pl.when(s < 10)
    def _():
        r0 = s * 1000
        pltpu.sync_copy(pn_sh.at[pl.ds(r0, 1000)], pn_out.at[c, pl.ds(r0, 1000)])
        pltpu.sync_copy(deg_sh.at[pl.ds(r0, 1000)], deg_out.at[c, pl.ds(r0, 1000)])


_phase_a = functools.partial(
    pl.kernel,
    out_type=(jax.ShapeDtypeStruct((NC, N, D_EDGE), jnp.float32),
              jax.ShapeDtypeStruct((NC, N, 8), jnp.float32)),
    mesh=_mesh,
    scratch_types=(
        pltpu.VMEM_SHARED((N, D_EDGE), jnp.float32),
        pltpu.VMEM_SHARED((N, 8), jnp.float32),
        pltpu.VMEM((CHA,), jnp.int32),
        pltpu.VMEM((CHA,), jnp.int32),
        pltpu.VMEM((CHA,), jnp.int32),
        pltpu.VMEM((CHA,), jnp.int32),
        pltpu.VMEM((CHA, D_EDGE), jnp.float32),
        pltpu.VMEM((CHA, D_EDGE), jnp.float32),
        pltpu.VMEM((CHA, 8), jnp.float32),
        pltpu.SemaphoreType.DMA,
        pltpu.SemaphoreType.DMA,
        pltpu.SemaphoreType.DMA,
        pltpu.SemaphoreType.DMA,
    ),
)(_phase_a_body)


def _phase_c_body(src_h, dst_h, g_h, z128_h,
                  agg_out,
                  agg_sh,
                  sidx0, sidx1, didx0, didx1, rows0, rows1,
                  gsem0, gsem1, isem0, isem1):
    c = lax.axis_index("c")
    s = lax.axis_index("s")

    @pl.when(s < 10)
    def _():
        r0 = s * 1000
        pltpu.sync_copy(z128_h.at[pl.ds(r0, 1000)], agg_sh.at[pl.ds(r0, 1000)])

    wid = c * NS + s
    plsc.subcore_barrier()

    pltpu.async_copy(src_h.at[wid, 0], sidx0, isem0)
    pltpu.async_copy(dst_h.at[wid, 0], didx0, isem0)
    pltpu.async_copy(src_h.at[wid, 1], sidx1, isem1)
    pltpu.async_copy(dst_h.at[wid, 1], didx1, isem1)
    pltpu.make_async_copy(src_h.at[wid, 0], sidx0, isem0).wait()
    pltpu.make_async_copy(dst_h.at[wid, 0], didx0, isem0).wait()
    pltpu.async_copy(g_h.at[sidx0], rows0, gsem0)

    @pl.loop(0, CHUNKS_C_MAIN, step=2)
    def _(i):
        pltpu.make_async_copy(g_h.at[sidx0], rows0, gsem0).wait()
        pltpu.make_async_copy(src_h.at[wid, i + 1], sidx1, isem1).wait()
        pltpu.make_async_copy(dst_h.at[wid, i + 1], didx1, isem1).wait()
        pltpu.async_copy(g_h.at[sidx1], rows1, gsem1)
        pltpu.sync_copy(rows0, agg_sh.at[didx0], add=True)

        @pl.when(i + 2 < CHUNKS_C)
        def _():
            pltpu.async_copy(src_h.at[wid, i + 2], sidx0, isem0)
            pltpu.async_copy(dst_h.at[wid, i + 2], didx0, isem0)

        pltpu.make_async_copy(g_h.at[sidx1], rows1, gsem1).wait()

        @pl.when(i + 2 < CHUNKS_C)
        def _():
            pltpu.make_async_copy(src_h.at[wid, i + 2], sidx0, isem0).wait()
            pltpu.make_async_copy(dst_h.at[wid, i + 2], didx0, isem0).wait()
            pltpu.async_copy(g_h.at[sidx0], rows0, gsem0)

        pltpu.sync_copy(rows1, agg_sh.at[didx1], add=True)

        @pl.when(i + 3 < CHUNKS_C)
        def _():
            pltpu.async_copy(src_h.at[wid, i + 3], sidx1, isem1)
            pltpu.async_copy(dst_h.at[wid, i + 3], didx1, isem1)

    pltpu.make_async_copy(g_h.at[sidx0], rows0, gsem0).wait()
    pltpu.sync_copy(rows0, agg_sh.at[didx0], add=True)

    plsc.subcore_barrier()

    @pl.when(s < 10)
    def _():
        r0 = s * 1000
        pltpu.sync_copy(agg_sh.at[pl.ds(r0, 1000)], agg_out.at[c, pl.ds(r0, 1000)])


_phase_c = functools.partial(
    pl.kernel,
    out_type=jax.ShapeDtypeStruct((NC, N, D_OUT), jnp.float32),
    mesh=_mesh,
    scratch_types=(
        pltpu.VMEM_SHARED((N, D_OUT), jnp.float32),
        pltpu.VMEM((CHC,), jnp.int32),
        pltpu.VMEM((CHC,), jnp.int32),
        pltpu.VMEM((CHC,), jnp.int32),
        pltpu.VMEM((CHC,), jnp.int32),
        pltpu.VMEM((CHC, D_OUT), jnp.float32),
        pltpu.VMEM((CHC, D_OUT), jnp.float32),
        pltpu.SemaphoreType.DMA,
        pltpu.SemaphoreType.DMA,
        pltpu.SemaphoreType.DMA,
        pltpu.SemaphoreType.DMA,
    ),
)(_phase_c_body)


_RB = 1000


def _dense1_body(x_ref, pn0_ref, pn1_ref, dg0_ref, dg1_ref,
                 We_ref, be_ref, W_ref, Wr_ref, bb_ref,
                 g_ref, base_ref):
    x = x_ref[...]
    pn = pn0_ref[...] + pn1_ref[...]
    deg = dg0_ref[...] + dg1_ref[...]
    dinv = lax.rsqrt(deg[:, :1] + 1.0)
    proj = jnp.dot(pn, We_ref[...], preferred_element_type=jnp.float32)
    proj = proj + be_ref[...]
    proj = jnp.where(proj >= 0, proj, 0.01 * proj)
    xcat = jnp.concatenate([x, proj], axis=1)
    h = jnp.dot(xcat, W_ref[...], preferred_element_type=jnp.float32)
    res = jnp.dot(xcat, Wr_ref[...], preferred_element_type=jnp.float32)
    g_ref[...] = h * dinv
    base_ref[...] = res + bb_ref[...]


def _dense1(x, pn0, pn1, dg0, dg1, We, be2, W, Wr, bb2):
    grid = (N // _RB,)
    return pl.pallas_call(
        _dense1_body,
        grid=grid,
        in_specs=[
            pl.BlockSpec((_RB, D_IN), lambda i: (i, 0)),
            pl.BlockSpec((_RB, D_EDGE), lambda i: (i, 0)),
            pl.BlockSpec((_RB, D_EDGE), lambda i: (i, 0)),
            pl.BlockSpec((_RB, 8), lambda i: (i, 0)),
            pl.BlockSpec((_RB, 8), lambda i: (i, 0)),
            pl.BlockSpec((D_EDGE, D_HALF), lambda i: (0, 0)),
            pl.BlockSpec((1, D_HALF), lambda i: (0, 0)),
            pl.BlockSpec((D_BASE, D_OUT), lambda i: (0, 0)),
            pl.BlockSpec((D_BASE, D_OUT), lambda i: (0, 0)),
            pl.BlockSpec((1, D_OUT), lambda i: (0, 0)),
        ],
        out_specs=[
            pl.BlockSpec((_RB, D_OUT), lambda i: (i, 0)),
            pl.BlockSpec((_RB, D_OUT), lambda i: (i, 0)),
        ],
        out_shape=[
            jax.ShapeDtypeStruct((N, D_OUT), jnp.float32),
            jax.ShapeDtypeStruct((N, D_OUT), jnp.float32),
        ],
    )(x, pn0, pn1, dg0, dg1, We, be2, W, Wr, bb2)


def _dense2_body(agg0_ref, agg1_ref, g_ref, base_ref,
                 dg0_ref, dg1_ref, out_ref):
    deg = dg0_ref[...] + dg1_ref[...]
    dinv = lax.rsqrt(deg[:, :1] + 1.0)
    agg = agg0_ref[...] + agg1_ref[...] + g_ref[...]
    out_ref[...] = jnp.maximum(dinv * agg + base_ref[...], 0.0)


def _dense2(agg0, agg1, g, base, dg0, dg1):
    grid = (N // _RB,)
    bspec = pl.BlockSpec((_RB, D_OUT), lambda i: (i, 0))
    return pl.pallas_call(
        _dense2_body,
        grid=grid,
        in_specs=[
            bspec, bspec, bspec, bspec,
            pl.BlockSpec((_RB, 8), lambda i: (i, 0)),
            pl.BlockSpec((_RB, 8), lambda i: (i, 0)),
        ],
        out_specs=bspec,
        out_shape=jax.ShapeDtypeStruct((N, D_OUT), jnp.float32),
    )(agg0, agg1, g, base, dg0, dg1)


@jax.jit
def _run(x, edge_index, edge_attr, We, be, W, b, Wr, br):
    src = edge_index[0]
    dst = edge_index[1]

    src_a = src.reshape(NW, CHUNKS_A, CHA)
    dst_a = dst.reshape(NW, CHUNKS_A, CHA)
    ea_a = edge_attr.reshape(NW, EDGES_TILE, D_EDGE)
    src_c = src.reshape(NW, CHUNKS_C, CHC)
    dst_c = dst.reshape(NW, CHUNKS_C, CHC)

    ones_v = jnp.ones((CHA, 8), jnp.float32)
    z16 = jnp.zeros((N, D_EDGE), jnp.float32)
    z8 = jnp.zeros((N, 8), jnp.float32)
    z128 = jnp.zeros((N, D_OUT), jnp.float32)

    pn_part, deg_part = _phase_a(src_a, dst_a, ea_a, ones_v, z16, z8)

    be2 = be.reshape(1, D_HALF)
    bb2 = (b + br).reshape(1, D_OUT)
    g, base = _dense1(x, pn_part[0], pn_part[1], deg_part[0], deg_part[1],
                      We, be2, W, Wr, bb2)

    agg = _phase_c(src_c, dst_c, g, z128)

    out = _dense2(agg[0], agg[1], g, base, deg_part[0], deg_part[1])
    return out


def kernel(x, edge_index, edge_attr, batch, We, be, W, b, Wr, br):
    return _run(x, edge_index, edge_attr, We, be, W, b, Wr, br)


# speedup vs baseline: 1.2654x; 1.0493x over previous
import functools

import jax
import jax.numpy as jnp
from jax import lax
from jax.experimental import pallas as pl
from jax.experimental.pallas import tpu as pltpu
from jax.experimental.pallas import tpu_sc as plsc

N = 10000
E = 320000
D_IN = 128
D_EDGE = 16
D_HALF = 64
D_BASE = 192
D_OUT = 128

NC = 2
NS = 16
NW = NC * NS
EDGES_TILE = E // NW

CHA = 200
CHUNKS_A = EDGES_TILE // CHA
CHC = 80
CHUNKS_C = EDGES_TILE // CHC
CHUNKS_C_MAIN = CHUNKS_C - 1

_mesh = plsc.VectorSubcoreMesh(
    core_axis_name="c", subcore_axis_name="s", num_cores=NC, num_subcores=NS)


def _phase_a_body(src_h, dst_h, ea_h, ones_h, z16_h, z8_h,
                  pn_out, deg_out,
                  pn_sh, deg_sh,
                  sidx0, sidx1, didx0, didx1, attr0, attr1, ones_v,
                  asem0, asem1, isem0, isem1):
    c = lax.axis_index("c")
    s = lax.axis_index("s")

    @pl.when(s < 10)
    def _():
        r0 = s * 1000
        pltpu.sync_copy(z16_h.at[pl.ds(r0, 1000)], pn_sh.at[pl.ds(r0, 1000)])
        pltpu.sync_copy(z8_h.at[pl.ds(r0, 1000)], deg_sh.at[pl.ds(r0, 1000)])

    wid = c * NS + s
    pltpu.sync_copy(ones_h, ones_v)
    plsc.subcore_barrier()

    pltpu.async_copy(src_h.at[wid, 0], sidx0, isem0)
    pltpu.async_copy(dst_h.at[wid, 0], didx0, isem0)
    pltpu.async_copy(src_h.at[wid, 1], sidx1, isem1)
    pltpu.async_copy(dst_h.at[wid, 1], didx1, isem1)
    pltpu.async_copy(ea_h.at[wid, pl.ds(0, CHA)], attr0, asem0)
    pltpu.async_copy(ea_h.at[wid, pl.ds(CHA, CHA)], attr1, asem1)

    @pl.loop(0, CHUNKS_A, step=2)
    def _(i):
        off = pl.multiple_of(i * CHA, 8)
        pltpu.make_async_copy(ea_h.at[wid, pl.ds(off, CHA)], attr0, asem0).wait()
        pltpu.make_async_copy(src_h.at[wid, i], sidx0, isem0).wait()
        pltpu.make_async_copy(dst_h.at[wid, i], didx0, isem0).wait()
        pltpu.sync_copy(attr0, pn_sh.at[sidx0], add=True)
        pltpu.sync_copy(ones_v, deg_sh.at[didx0], add=True)

        @pl.when(i + 2 < CHUNKS_A)
        def _():
            pltpu.async_copy(src_h.at[wid, i + 2], sidx0, isem0)
            pltpu.async_copy(dst_h.at[wid, i + 2], didx0, isem0)
            pltpu.async_copy(ea_h.at[wid, pl.ds(off + 2 * CHA, CHA)], attr0, asem0)

        pltpu.make_async_copy(ea_h.at[wid, pl.ds(off + CHA, CHA)], attr1, asem1).wait()
        pltpu.make_async_copy(src_h.at[wid, i + 1], sidx1, isem1).wait()
        pltpu.make_async_copy(dst_h.at[wid, i + 1], didx1, isem1).wait()
        pltpu.sync_copy(attr1, pn_sh.at[sidx1], add=True)
        pltpu.sync_copy(ones_v, deg_sh.at[didx1], add=True)

        @pl.when(i + 3 < CHUNKS_A)
        def _():
            pltpu.async_copy(src_h.at[wid, i + 3], sidx1, isem1)
            pltpu.async_copy(dst_h.at[wid, i + 3], didx1, isem1)
            pltpu.async_copy(ea_h.at[wid, pl.ds(off + 3 * CHA, CHA)], attr1, asem1)

    plsc.subcore_barrier()

    @pl.when(s < 10)
    def _():
        r0 = s * 1000
        pltpu.sync_copy(pn_sh.at[pl.ds(r0, 1000)], pn_out.at[c, pl.ds(r0, 1000)])
        pltpu.sync_copy(deg_sh.at[pl.ds(r0, 1000)], deg_out.at[c, pl.ds(r0, 1000)])


_phase_a = functools.partial(
    pl.kernel,
    out_type=(jax.ShapeDtypeStruct((NC, N, D_EDGE), jnp.float32),
              jax.ShapeDtypeStruct((NC, N, 8), jnp.float32)),
    mesh=_mesh,
    scratch_types=(
        pltpu.VMEM_SHARED((N, D_EDGE), jnp.float32),
        pltpu.VMEM_SHARED((N, 8), jnp.float32),
        pltpu.VMEM((CHA,), jnp.int32),
        pltpu.VMEM((CHA,), jnp.int32),
        pltpu.VMEM((CHA,), jnp.int32),
        pltpu.VMEM((CHA,), jnp.int32),
        pltpu.VMEM((CHA, D_EDGE), jnp.float32),
        pltpu.VMEM((CHA, D_EDGE), jnp.float32),
        pltpu.VMEM((CHA, 8), jnp.float32),
        pltpu.SemaphoreType.DMA,
        pltpu.SemaphoreType.DMA,
        pltpu.SemaphoreType.DMA,
        pltpu.SemaphoreType.DMA,
    ),
)(_phase_a_body)


def _phase_c_body(src_h, dst_h, g_h, z128_h,
                  agg_out,
                  agg_sh,
                  sidx0, sidx1, sidx2, sidx3,
                  didx0, didx1, didx2, didx3,
                  rows0, rows1, rows2, rows3,
                  gsem0, gsem1, gsem2, gsem3,
                  isem0, isem1, isem2, isem3):
    c = lax.axis_index("c")
    s = lax.axis_index("s")

    @pl.when(s < 10)
    def _():
        r0 = s * 1000
        pltpu.sync_copy(z128_h.at[pl.ds(r0, 1000)], agg_sh.at[pl.ds(r0, 1000)])

    wid = c * NS + s
    plsc.subcore_barrier()

    sidx = (sidx0, sidx1, sidx2, sidx3)
    didx = (didx0, didx1, didx2, didx3)
    rows = (rows0, rows1, rows2, rows3)
    gsem = (gsem0, gsem1, gsem2, gsem3)
    isem = (isem0, isem1, isem2, isem3)

    for b in range(4):
        pltpu.async_copy(src_h.at[wid, b], sidx[b], isem[b])
        pltpu.async_copy(dst_h.at[wid, b], didx[b], isem[b])
    for b in range(3):
        pltpu.make_async_copy(src_h.at[wid, b], sidx[b], isem[b]).wait()
        pltpu.make_async_copy(dst_h.at[wid, b], didx[b], isem[b]).wait()
        pltpu.async_copy(g_h.at[sidx[b]], rows[b], gsem[b])

    @pl.loop(0, CHUNKS_C_MAIN, step=4)
    def _(i):
        for b in range(4):
            k = i + b
            b3 = (b + 3) % 4
            pltpu.make_async_copy(g_h.at[sidx[b]], rows[b], gsem[b]).wait()

            @pl.when(k + 3 < CHUNKS_C)
            def _():
                pltpu.make_async_copy(src_h.at[wid, k + 3], sidx[b3], isem[b3]).wait()
                pltpu.make_async_copy(dst_h.at[wid, k + 3], didx[b3], isem[b3]).wait()
                pltpu.async_copy(g_h.at[sidx[b3]], rows[b3], gsem[b3])

            pltpu.sync_copy(rows[b], agg_sh.at[didx[b]], add=True)

            @pl.when(k + 4 < CHUNKS_C)
            def _():
                pltpu.async_copy(src_h.at[wid, k + 4], sidx[b], isem[b])
                pltpu.async_copy(dst_h.at[wid, k + 4], didx[b], isem[b])

    pltpu.make_async_copy(g_h.at[sidx[0]], rows[0], gsem[0]).wait()
    pltpu.sync_copy(rows[0], agg_sh.at[didx[0]], add=True)

    plsc.subcore_barrier()

    @pl.when(s < 10)
    def _():
        r0 = s * 1000
        pltpu.sync_copy(agg_sh.at[pl.ds(r0, 1000)], agg_out.at[c, pl.ds(r0, 1000)])


_phase_c = functools.partial(
    pl.kernel,
    out_type=jax.ShapeDtypeStruct((NC, N, D_OUT), jnp.float32),
    mesh=_mesh,
    scratch_types=(
        pltpu.VMEM_SHARED((N, D_OUT), jnp.float32),
        pltpu.VMEM((CHC,), jnp.int32),
        pltpu.VMEM((CHC,), jnp.int32),
        pltpu.VMEM((CHC,), jnp.int32),
        pltpu.VMEM((CHC,), jnp.int32),
        pltpu.VMEM((CHC,), jnp.int32),
        pltpu.VMEM((CHC,), jnp.int32),
        pltpu.VMEM((CHC,), jnp.int32),
        pltpu.VMEM((CHC,), jnp.int32),
        pltpu.VMEM((CHC, D_OUT), jnp.float32),
        pltpu.VMEM((CHC, D_OUT), jnp.float32),
        pltpu.VMEM((CHC, D_OUT), jnp.float32),
        pltpu.VMEM((CHC, D_OUT), jnp.float32),
        pltpu.SemaphoreType.DMA,
        pltpu.SemaphoreType.DMA,
        pltpu.SemaphoreType.DMA,
        pltpu.SemaphoreType.DMA,
        pltpu.SemaphoreType.DMA,
        pltpu.SemaphoreType.DMA,
        pltpu.SemaphoreType.DMA,
        pltpu.SemaphoreType.DMA,
    ),
)(_phase_c_body)


_RB = 1000


def _dense1_body(x_ref, pn0_ref, pn1_ref, dg0_ref, dg1_ref,
                 We_ref, be_ref, W_ref, Wr_ref, bb_ref,
                 g_ref, base_ref):
    x = x_ref[...]
    pn = pn0_ref[...] + pn1_ref[...]
    deg = dg0_ref[...] + dg1_ref[...]
    dinv = lax.rsqrt(deg[:, :1] + 1.0)
    proj = jnp.dot(pn, We_ref[...], preferred_element_type=jnp.float32)
    proj = proj + be_ref[...]
    proj = jnp.where(proj >= 0, proj, 0.01 * proj)
    xcat = jnp.concatenate([x, proj], axis=1)
    h = jnp.dot(xcat, W_ref[...], preferred_element_type=jnp.float32)
    res = jnp.dot(xcat, Wr_ref[...], preferred_element_type=jnp.float32)
    g_ref[...] = h * dinv
    base_ref[...] = res + bb_ref[...]


def _dense1(x, pn0, pn1, dg0, dg1, We, be2, W, Wr, bb2):
    grid = (N // _RB,)
    return pl.pallas_call(
        _dense1_body,
        grid=grid,
        in_specs=[
            pl.BlockSpec((_RB, D_IN), lambda i: (i, 0)),
            pl.BlockSpec((_RB, D_EDGE), lambda i: (i, 0)),
            pl.BlockSpec((_RB, D_EDGE), lambda i: (i, 0)),
            pl.BlockSpec((_RB, 8), lambda i: (i, 0)),
            pl.BlockSpec((_RB, 8), lambda i: (i, 0)),
            pl.BlockSpec((D_EDGE, D_HALF), lambda i: (0, 0)),
            pl.BlockSpec((1, D_HALF), lambda i: (0, 0)),
            pl.BlockSpec((D_BASE, D_OUT), lambda i: (0, 0)),
            pl.BlockSpec((D_BASE, D_OUT), lambda i: (0, 0)),
            pl.BlockSpec((1, D_OUT), lambda i: (0, 0)),
        ],
        out_specs=[
            pl.BlockSpec((_RB, D_OUT), lambda i: (i, 0)),
            pl.BlockSpec((_RB, D_OUT), lambda i: (i, 0)),
        ],
        out_shape=[
            jax.ShapeDtypeStruct((N, D_OUT), jnp.float32),
            jax.ShapeDtypeStruct((N, D_OUT), jnp.float32),
        ],
    )(x, pn0, pn1, dg0, dg1, We, be2, W, Wr, bb2)


def _dense2_body(agg0_ref, agg1_ref, g_ref, base_ref,
                 dg0_ref, dg1_ref, out_ref):
    deg = dg0_ref[...] + dg1_ref[...]
    dinv = lax.rsqrt(deg[:, :1] + 1.0)
    agg = agg0_ref[...] + agg1_ref[...] + g_ref[...]
    out_ref[...] = jnp.maximum(dinv * agg + base_ref[...], 0.0)


def _dense2(agg0, agg1, g, base, dg0, dg1):
    grid = (N // _RB,)
    bspec = pl.BlockSpec((_RB, D_OUT), lambda i: (i, 0))
    return pl.pallas_call(
        _dense2_body,
        grid=grid,
        in_specs=[
            bspec, bspec, bspec, bspec,
            pl.BlockSpec((_RB, 8), lambda i: (i, 0)),
            pl.BlockSpec((_RB, 8), lambda i: (i, 0)),
        ],
        out_specs=bspec,
        out_shape=jax.ShapeDtypeStruct((N, D_OUT), jnp.float32),
    )(agg0, agg1, g, base, dg0, dg1)


@jax.jit
def _run(x, edge_index, edge_attr, We, be, W, b, Wr, br):
    src = edge_index[0]
    dst = edge_index[1]

    src_a = src.reshape(NW, CHUNKS_A, CHA)
    dst_a = dst.reshape(NW, CHUNKS_A, CHA)
    ea_a = edge_attr.reshape(NW, EDGES_TILE, D_EDGE)
    src_c = src.reshape(NW, CHUNKS_C, CHC)
    dst_c = dst.reshape(NW, CHUNKS_C, CHC)

    ones_v = jnp.ones((CHA, 8), jnp.float32)
    z16 = jnp.zeros((N, D_EDGE), jnp.float32)
    z8 = jnp.zeros((N, 8), jnp.float32)
    z128 = jnp.zeros((N, D_OUT), jnp.float32)

    pn_part, deg_part = _phase_a(src_a, dst_a, ea_a, ones_v, z16, z8)

    be2 = be.reshape(1, D_HALF)
    bb2 = (b + br).reshape(1, D_OUT)
    g, base = _dense1(x, pn_part[0], pn_part[1], deg_part[0], deg_part[1],
                      We, be2, W, Wr, bb2)

    agg = _phase_c(src_c, dst_c, g, z128)

    out = _dense2(agg[0], agg[1], g, base, deg_part[0], deg_part[1])
    return out


def kernel(x, edge_index, edge_attr, batch, We, be, W, b, Wr, br):
    return _run(x, edge_index, edge_attr, We, be, W, b, Wr, br)


# speedup vs baseline: 1.4401x; 1.1380x over previous
import functools

import jax
import jax.numpy as jnp
from jax import lax
from jax.experimental import pallas as pl
from jax.experimental.pallas import tpu as pltpu
from jax.experimental.pallas import tpu_sc as plsc

N = 10000
E = 320000
D_IN = 128
D_EDGE = 16
D_HALF = 64
D_BASE = 192
D_OUT = 128

NC = 2
NS = 16
NW = NC * NS
EDGES_TILE = E // NW

CHA = 200
CHUNKS_A = EDGES_TILE // CHA
CHC = 80
CHUNKS_C = EDGES_TILE // CHC
CHUNKS_C_MAIN = CHUNKS_C - 1

_mesh = plsc.VectorSubcoreMesh(
    core_axis_name="c", subcore_axis_name="s", num_cores=NC, num_subcores=NS)


def _phase_a_body(src_h, dst_h, ea_h, ones_h, z16_h, z8_h,
                  pn_out, deg_out,
                  pn_sh, deg_sh,
                  sidx0, sidx1, didx0, didx1, attr0, attr1, ones_v,
                  asem0, asem1, isem0, isem1):
    c = lax.axis_index("c")
    s = lax.axis_index("s")

    @pl.when(s < 10)
    def _():
        r0 = s * 1000
        pltpu.sync_copy(z16_h.at[pl.ds(r0, 1000)], pn_sh.at[pl.ds(r0, 1000)])
        pltpu.sync_copy(z8_h.at[pl.ds(r0, 1000)], deg_sh.at[pl.ds(r0, 1000)])

    wid = c * NS + s
    pltpu.sync_copy(ones_h, ones_v)
    plsc.subcore_barrier()

    pltpu.async_copy(src_h.at[wid, 0], sidx0, isem0)
    pltpu.async_copy(dst_h.at[wid, 0], didx0, isem0)
    pltpu.async_copy(src_h.at[wid, 1], sidx1, isem1)
    pltpu.async_copy(dst_h.at[wid, 1], didx1, isem1)
    pltpu.async_copy(ea_h.at[wid, pl.ds(0, CHA)], attr0, asem0)
    pltpu.async_copy(ea_h.at[wid, pl.ds(CHA, CHA)], attr1, asem1)

    @pl.loop(0, CHUNKS_A, step=2)
    def _(i):
        off = pl.multiple_of(i * CHA, 8)
        pltpu.make_async_copy(ea_h.at[wid, pl.ds(off, CHA)], attr0, asem0).wait()
        pltpu.make_async_copy(src_h.at[wid, i], sidx0, isem0).wait()
        pltpu.make_async_copy(dst_h.at[wid, i], didx0, isem0).wait()
        pltpu.sync_copy(attr0, pn_sh.at[sidx0], add=True)
        pltpu.sync_copy(ones_v, deg_sh.at[didx0], add=True)

        @pl.when(i + 2 < CHUNKS_A)
        def _():
            pltpu.async_copy(src_h.at[wid, i + 2], sidx0, isem0)
            pltpu.async_copy(dst_h.at[wid, i + 2], didx0, isem0)
            pltpu.async_copy(ea_h.at[wid, pl.ds(off + 2 * CHA, CHA)], attr0, asem0)

        pltpu.make_async_copy(ea_h.at[wid, pl.ds(off + CHA, CHA)], attr1, asem1).wait()
        pltpu.make_async_copy(src_h.at[wid, i + 1], sidx1, isem1).wait()
        pltpu.make_async_copy(dst_h.at[wid, i + 1], didx1, isem1).wait()
        pltpu.sync_copy(attr1, pn_sh.at[sidx1], add=True)
        pltpu.sync_copy(ones_v, deg_sh.at[didx1], add=True)

        @pl.when(i + 3 < CHUNKS_A)
        def _():
            pltpu.async_copy(src_h.at[wid, i + 3], sidx1, isem1)
            pltpu.async_copy(dst_h.at[wid, i + 3], didx1, isem1)
            pltpu.async_copy(ea_h.at[wid, pl.ds(off + 3 * CHA, CHA)], attr1, asem1)

    plsc.subcore_barrier()

    @pl.when(s < 10)
    def _():
        r0 = s * 1000
        pltpu.sync_copy(pn_sh.at[pl.ds(r0, 1000)], pn_out.at[c, pl.ds(r0, 1000)])
        pltpu.sync_copy(deg_sh.at[pl.ds(r0, 1000)], deg_out.at[c, pl.ds(r0, 1000)])


_phase_a = functools.partial(
    pl.kernel,
    out_type=(jax.ShapeDtypeStruct((NC, N, D_EDGE), jnp.float32),
              jax.ShapeDtypeStruct((NC, N, 8), jnp.float32)),
    mesh=_mesh,
    scratch_types=(
        pltpu.VMEM_SHARED((N, D_EDGE), jnp.float32),
        pltpu.VMEM_SHARED((N, 8), jnp.float32),
        pltpu.VMEM((CHA,), jnp.int32),
        pltpu.VMEM((CHA,), jnp.int32),
        pltpu.VMEM((CHA,), jnp.int32),
        pltpu.VMEM((CHA,), jnp.int32),
        pltpu.VMEM((CHA, D_EDGE), jnp.float32),
        pltpu.VMEM((CHA, D_EDGE), jnp.float32),
        pltpu.VMEM((CHA, 8), jnp.float32),
        pltpu.SemaphoreType.DMA,
        pltpu.SemaphoreType.DMA,
        pltpu.SemaphoreType.DMA,
        pltpu.SemaphoreType.DMA,
    ),
)(_phase_a_body)


def _phase_c_body(src_h, dst_h, g_h, z128_h,
                  agg_out,
                  agg_sh,
                  sidx0, sidx1, sidx2, sidx3,
                  didx0, didx1, didx2, didx3,
                  sdidx0, sdidx1, sdidx2, sdidx3,
                  rows0, rows1, rows2, rows3,
                  gsem0, gsem1, gsem2, gsem3,
                  isem0, isem1, isem2, isem3,
                  ssem0, ssem1, ssem2, ssem3):
    c = lax.axis_index("c")
    s = lax.axis_index("s")

    @pl.when(s < 10)
    def _():
        r0 = s * 1000
        pltpu.sync_copy(z128_h.at[pl.ds(r0, 1000)], agg_sh.at[pl.ds(r0, 1000)])

    wid = c * NS + s
    plsc.subcore_barrier()

    sidx = (sidx0, sidx1, sidx2, sidx3)
    didx = (didx0, didx1, didx2, didx3)
    sdidx = (sdidx0, sdidx1, sdidx2, sdidx3)
    rows = (rows0, rows1, rows2, rows3)
    gsem = (gsem0, gsem1, gsem2, gsem3)
    isem = (isem0, isem1, isem2, isem3)
    ssem = (ssem0, ssem1, ssem2, ssem3)

    def wait_scatter(b):
        pltpu.make_async_copy(rows[b], agg_sh.at[sdidx[b]], ssem[b]).wait()

    for b in range(4):
        pltpu.async_copy(src_h.at[wid, b], sidx[b], isem[b])
        pltpu.async_copy(dst_h.at[wid, b], didx[b], isem[b])
    for b in range(3):
        pltpu.make_async_copy(src_h.at[wid, b], sidx[b], isem[b]).wait()
        pltpu.make_async_copy(dst_h.at[wid, b], didx[b], isem[b]).wait()
        pltpu.async_copy(g_h.at[sidx[b]], rows[b], gsem[b])

    @pl.loop(0, CHUNKS_C_MAIN, step=4)
    def _(i):
        for b in range(4):
            k = i + b
            b3 = (b + 3) % 4
            pltpu.make_async_copy(g_h.at[sidx[b]], rows[b], gsem[b]).wait()

            if b == 0:
                @pl.when(i > 0)
                def _():
                    wait_scatter(b3)
            else:
                wait_scatter(b3)

            @pl.when(k + 3 < CHUNKS_C)
            def _():
                pltpu.make_async_copy(src_h.at[wid, k + 3], sidx[b3], isem[b3]).wait()
                pltpu.make_async_copy(dst_h.at[wid, k + 3], didx[b3], isem[b3]).wait()
                pltpu.async_copy(g_h.at[sidx[b3]], rows[b3], gsem[b3])

            for t in range(CHC // 16):
                sdidx[b][pl.ds(t * 16, 16)] = didx[b][pl.ds(t * 16, 16)]
            pltpu.async_copy(rows[b], agg_sh.at[sdidx[b]], ssem[b], add=True)

            @pl.when(k + 4 < CHUNKS_C)
            def _():
                pltpu.async_copy(src_h.at[wid, k + 4], sidx[b], isem[b])
                pltpu.async_copy(dst_h.at[wid, k + 4], didx[b], isem[b])

    pltpu.make_async_copy(g_h.at[sidx[0]], rows[0], gsem[0]).wait()
    wait_scatter(3)
    for t in range(CHC // 16):
        sdidx[0][pl.ds(t * 16, 16)] = didx[0][pl.ds(t * 16, 16)]
    pltpu.async_copy(rows[0], agg_sh.at[sdidx[0]], ssem[0], add=True)
    wait_scatter(0)

    plsc.subcore_barrier()

    @pl.when(s < 10)
    def _():
        r0 = s * 1000
        pltpu.sync_copy(agg_sh.at[pl.ds(r0, 1000)], agg_out.at[c, pl.ds(r0, 1000)])


_phase_c = functools.partial(
    pl.kernel,
    out_type=jax.ShapeDtypeStruct((NC, N, D_OUT), jnp.float32),
    mesh=_mesh,
    scratch_types=(
        (pltpu.VMEM_SHARED((N, D_OUT), jnp.float32),)
        + tuple(pltpu.VMEM((CHC,), jnp.int32) for _ in range(12))
        + tuple(pltpu.VMEM((CHC, D_OUT), jnp.float32) for _ in range(4))
        + tuple(pltpu.SemaphoreType.DMA for _ in range(12))
    ),
)(_phase_c_body)


_RB = 1000


def _dense1_body(x_ref, pn0_ref, pn1_ref, dg0_ref, dg1_ref,
                 We_ref, be_ref, W_ref, Wr_ref, bb_ref,
                 g_ref, base_ref):
    x = x_ref[...]
    pn = pn0_ref[...] + pn1_ref[...]
    deg = dg0_ref[...] + dg1_ref[...]
    dinv = lax.rsqrt(deg[:, :1] + 1.0)
    proj = jnp.dot(pn, We_ref[...], preferred_element_type=jnp.float32)
    proj = proj + be_ref[...]
    proj = jnp.where(proj >= 0, proj, 0.01 * proj)
    xcat = jnp.concatenate([x, proj], axis=1)
    h = jnp.dot(xcat, W_ref[...], preferred_element_type=jnp.float32)
    res = jnp.dot(xcat, Wr_ref[...], preferred_element_type=jnp.float32)
    g_ref[...] = h * dinv
    base_ref[...] = res + bb_ref[...]


def _dense1(x, pn0, pn1, dg0, dg1, We, be2, W, Wr, bb2):
    grid = (N // _RB,)
    return pl.pallas_call(
        _dense1_body,
        grid=grid,
        in_specs=[
            pl.BlockSpec((_RB, D_IN), lambda i: (i, 0)),
            pl.BlockSpec((_RB, D_EDGE), lambda i: (i, 0)),
            pl.BlockSpec((_RB, D_EDGE), lambda i: (i, 0)),
            pl.BlockSpec((_RB, 8), lambda i: (i, 0)),
            pl.BlockSpec((_RB, 8), lambda i: (i, 0)),
            pl.BlockSpec((D_EDGE, D_HALF), lambda i: (0, 0)),
            pl.BlockSpec((1, D_HALF), lambda i: (0, 0)),
            pl.BlockSpec((D_BASE, D_OUT), lambda i: (0, 0)),
            pl.BlockSpec((D_BASE, D_OUT), lambda i: (0, 0)),
            pl.BlockSpec((1, D_OUT), lambda i: (0, 0)),
        ],
        out_specs=[
            pl.BlockSpec((_RB, D_OUT), lambda i: (i, 0)),
            pl.BlockSpec((_RB, D_OUT), lambda i: (i, 0)),
        ],
        out_shape=[
            jax.ShapeDtypeStruct((N, D_OUT), jnp.float32),
            jax.ShapeDtypeStruct((N, D_OUT), jnp.float32),
        ],
    )(x, pn0, pn1, dg0, dg1, We, be2, W, Wr, bb2)


def _dense2_body(agg0_ref, agg1_ref, g_ref, base_ref,
                 dg0_ref, dg1_ref, out_ref):
    deg = dg0_ref[...] + dg1_ref[...]
    dinv = lax.rsqrt(deg[:, :1] + 1.0)
    agg = agg0_ref[...] + agg1_ref[...] + g_ref[...]
    out_ref[...] = jnp.maximum(dinv * agg + base_ref[...], 0.0)


def _dense2(agg0, agg1, g, base, dg0, dg1):
    grid = (N // _RB,)
    bspec = pl.BlockSpec((_RB, D_OUT), lambda i: (i, 0))
    return pl.pallas_call(
        _dense2_body,
        grid=grid,
        in_specs=[
            bspec, bspec, bspec, bspec,
            pl.BlockSpec((_RB, 8), lambda i: (i, 0)),
            pl.BlockSpec((_RB, 8), lambda i: (i, 0)),
        ],
        out_specs=bspec,
        out_shape=jax.ShapeDtypeStruct((N, D_OUT), jnp.float32),
    )(agg0, agg1, g, base, dg0, dg1)


@jax.jit
def _run(x, edge_index, edge_attr, We, be, W, b, Wr, br):
    src = edge_index[0]
    dst = edge_index[1]

    src_a = src.reshape(NW, CHUNKS_A, CHA)
    dst_a = dst.reshape(NW, CHUNKS_A, CHA)
    ea_a = edge_attr.reshape(NW, EDGES_TILE, D_EDGE)
    src_c = src.reshape(NW, CHUNKS_C, CHC)
    dst_c = dst.reshape(NW, CHUNKS_C, CHC)

    ones_v = jnp.ones((CHA, 8), jnp.float32)
    z16 = jnp.zeros((N, D_EDGE), jnp.float32)
    z8 = jnp.zeros((N, 8), jnp.float32)
    z128 = jnp.zeros((N, D_OUT), jnp.float32)

    pn_part, deg_part = _phase_a(src_a, dst_a, ea_a, ones_v, z16, z8)

    be2 = be.reshape(1, D_HALF)
    bb2 = (b + br).reshape(1, D_OUT)
    g, base = _dense1(x, pn_part[0], pn_part[1], deg_part[0], deg_part[1],
                      We, be2, W, Wr, bb2)

    agg = _phase_c(src_c, dst_c, g, z128)

    out = _dense2(agg[0], agg[1], g, base, deg_part[0], deg_part[1])
    return out


def kernel(x, edge_index, edge_attr, batch, We, be, W, b, Wr, br):
    return _run(x, edge_index, edge_attr, We, be, W, b, Wr, br)
